# Initial kernel scaffold; baseline (speedup 1.0000x reference)
import jax
import jax.numpy as jnp
from jax.experimental import pallas as pl


def kernel(x, edge_index, batch_size, W_init, W_lin0, b_lin0, W_ih0, W_hh0, b_ih0, b_hh0, W_lin1, b_lin1, W_ih1, W_hh1, b_ih1, b_hh1):
    raise NotImplementedError("write your pallas kernel here")


# speedup vs baseline: 5.6097x; 5.6097x over previous
import functools

import jax
import jax.numpy as jnp
from jax import lax
from jax.experimental import pallas as pl
from jax.experimental.pallas import tpu as pltpu
from jax.experimental.pallas import tpu_sc as plsc

_N = 10000
_BATCH = 100
_E = 320000
_D = 128
_NPAD = 10240

_NC = 2
_NS = 16
_NW = _NC * _NS
_CH = 128
_NCHUNK = _E // _CH
_CH_PER_W = _NCHUNK // _NW
_CH_EXTRA = _NCHUNK % _NW


def _make_agg():
  mesh = plsc.VectorSubcoreMesh(core_axis_name="c", subcore_axis_name="s")
  rows_per_tile = _NPAD // _NS

  @functools.partial(
      pl.kernel,
      mesh=mesh,
      out_type=jax.ShapeDtypeStruct((_NC, _NPAD, _D), jnp.float32),
      scratch_types=[
          pltpu.VMEM((_CH,), jnp.int32),
          pltpu.VMEM((_CH,), jnp.int32),
          pltpu.VMEM((_CH, _D), jnp.float32),
          pltpu.VMEM((16, _D), jnp.float32),
          pltpu.VMEM_SHARED((_NPAD, _D), jnp.float32),
          pltpu.SemaphoreType.DMA,
      ],
  )
  def agg(feats_hbm, src_hbm, dst_hbm, out_hbm,
          src_v, dst_v, rows_v, zero_v, acc_sh, sem):
    cid = lax.axis_index("c")
    sid = lax.axis_index("s")
    wid = sid * _NC + cid

    z16 = jnp.zeros((16,), jnp.float32)
    for r in range(16):
      for c in range(_D // 16):
        zero_v[r, pl.ds(c * 16, 16)] = z16

    def zero_body(i, carry):
      pltpu.sync_copy(zero_v, acc_sh.at[pl.ds(sid * rows_per_tile + i * 16, 16)])
      return carry
    lax.fori_loop(0, rows_per_tile // 16, zero_body, 0)
    plsc.subcore_barrier()

    def do_chunk(chunk):
      off = chunk * _CH
      pltpu.sync_copy(src_hbm.at[pl.ds(off, _CH)], src_v)
      pltpu.async_copy(feats_hbm.at[src_v], rows_v, sem).wait()
      pltpu.sync_copy(dst_hbm.at[pl.ds(off, _CH)], dst_v)
      pltpu.sync_copy(rows_v, acc_sh.at[dst_v], add=True)

    def chunk_body(c, carry):
      do_chunk(wid + c * _NW)
      return carry
    lax.fori_loop(0, _CH_PER_W, chunk_body, 0)

    @pl.when(wid < _CH_EXTRA)
    def _():
      do_chunk(wid + _CH_PER_W * _NW)

    plsc.subcore_barrier()

    pltpu.sync_copy(
        acc_sh.at[pl.ds(sid * rows_per_tile, rows_per_tile)],
        out_hbm.at[cid, pl.ds(sid * rows_per_tile, rows_per_tile)])

  return agg


_agg = _make_agg()

_R = 1000


def _init_transform(x, w_t):
  def body(x_ref, w_ref, o_ref):
    o_ref[...] = jnp.dot(x_ref[...], w_ref[...],
                         preferred_element_type=jnp.float32)
  return pl.pallas_call(
      body,
      grid=(_N // _R,),
      in_specs=[
          pl.BlockSpec((_R, _D), lambda i: (i, 0)),
          pl.BlockSpec((_D, _D), lambda i: (0, 0)),
      ],
      out_specs=pl.BlockSpec((_R, _D), lambda i: (i, 0)),
      out_shape=jax.ShapeDtypeStruct((_N, _D), jnp.float32),
  )(x, w_t)


def _gru(parts, feats, wl_t, bl, wi_t, wh_t, bi, bh):
  def body(p_ref, f_ref, wl_ref, bl_ref, wi_ref, wh_ref, bi_ref, bh_ref,
           o_ref):
    agg = p_ref[0] + p_ref[1]
    f = f_ref[...]
    h = jnp.dot(agg, wl_ref[...], preferred_element_type=jnp.float32) + bl_ref[...]
    gi = jnp.dot(h, wi_ref[...], preferred_element_type=jnp.float32) + bi_ref[...]
    gh = jnp.dot(f, wh_ref[...], preferred_element_type=jnp.float32) + bh_ref[...]
    r = jax.nn.sigmoid(gi[:, :_D] + gh[:, :_D])
    z = jax.nn.sigmoid(gi[:, _D:2 * _D] + gh[:, _D:2 * _D])
    n = jnp.tanh(gi[:, 2 * _D:] + r * gh[:, 2 * _D:])
    o_ref[...] = (1.0 - z) * n + z * f

  return pl.pallas_call(
      body,
      grid=(_N // _R,),
      in_specs=[
          pl.BlockSpec((_NC, _R, _D), lambda i: (0, i, 0)),
          pl.BlockSpec((_R, _D), lambda i: (i, 0)),
          pl.BlockSpec((_D, _D), lambda i: (0, 0)),
          pl.BlockSpec((1, _D), lambda i: (0, 0)),
          pl.BlockSpec((_D, 3 * _D), lambda i: (0, 0)),
          pl.BlockSpec((_D, 3 * _D), lambda i: (0, 0)),
          pl.BlockSpec((1, 3 * _D), lambda i: (0, 0)),
          pl.BlockSpec((1, 3 * _D), lambda i: (0, 0)),
      ],
      out_specs=pl.BlockSpec((_R, _D), lambda i: (i, 0)),
      out_shape=jax.ShapeDtypeStruct((_N, _D), jnp.float32),
  )(parts, feats, wl_t, bl, wi_t, wh_t, bi, bh)


def kernel(x, edge_index, batch_size, W_init,
           W_lin0, b_lin0, W_ih0, W_hh0, b_ih0, b_hh0,
           W_lin1, b_lin1, W_ih1, W_hh1, b_ih1, b_hh1):
  src = edge_index[0].astype(jnp.int32)
  dst = edge_index[1].astype(jnp.int32)

  feats = _init_transform(x, W_init.T)

  parts = _agg(feats, src, dst)
  feats = _gru(parts, feats, W_lin0.T, b_lin0.reshape(1, -1),
               W_ih0.T, W_hh0.T, b_ih0.reshape(1, -1), b_hh0.reshape(1, -1))

  parts = _agg(feats, src, dst)
  feats = _gru(parts, feats, W_lin1.T, b_lin1.reshape(1, -1),
               W_ih1.T, W_hh1.T, b_ih1.reshape(1, -1), b_hh1.reshape(1, -1))

  return feats.reshape(_BATCH, -1, _D)


# speedup vs baseline: 8.0950x; 1.4430x over previous
import functools

import jax
import jax.numpy as jnp
from jax import lax
from jax.experimental import pallas as pl
from jax.experimental.pallas import tpu as pltpu
from jax.experimental.pallas import tpu_sc as plsc

_N = 10000
_BATCH = 100
_E = 320000
_D = 128
_NPAD = 10240

_NC = 2
_NS = 16
_NW = _NC * _NS
_CH = 128
_CHW = 80
_EPAD = _CHW * _CH * _NW


def _make_agg():
  mesh = plsc.VectorSubcoreMesh(core_axis_name="c", subcore_axis_name="s")
  rows_per_tile = _NPAD // _NS
  _ZR = 64

  @functools.partial(
      pl.kernel,
      mesh=mesh,
      out_type=jax.ShapeDtypeStruct((_NC, _NPAD, _D), jnp.float32),
      scratch_types=[
          pltpu.VMEM((2, _CH), jnp.int32),
          pltpu.VMEM((2, _CH), jnp.int32),
          pltpu.VMEM((_CH, _D), jnp.float32),
          pltpu.VMEM((_CH, _D), jnp.float32),
          pltpu.VMEM((_ZR, _D), jnp.float32),
          pltpu.VMEM_SHARED((_NPAD, _D), jnp.float32),
          pltpu.SemaphoreType.DMA,
          pltpu.SemaphoreType.DMA,
          pltpu.SemaphoreType.DMA,
          pltpu.SemaphoreType.DMA,
          pltpu.SemaphoreType.DMA,
          pltpu.SemaphoreType.DMA,
          pltpu.SemaphoreType.DMA,
      ],
  )
  def agg(feats_hbm, edge_hbm, out_hbm,
          idx0, idx1, rows0, rows1, zero_v, acc_sh,
          si0, si1, sg0, sg1, ss0, ss1, sz):
    cid = lax.axis_index("c")
    sid = lax.axis_index("s")
    wid = sid * _NC + cid

    idxs = (idx0, idx1)
    rows = (rows0, rows1)
    sem_i = (si0, si1)
    sem_g = (sg0, sg1)
    sem_s = (ss0, ss1)

    def chunk_off(s):
      return (wid + s * _NW) * _CH

    def issue_idx(b, off):
      pltpu.async_copy(edge_hbm.at[:, pl.ds(off, _CH)], idxs[b], sem_i[b])

    def wait_idx(b):
      pltpu.make_async_copy(
          edge_hbm.at[:, pl.ds(0, _CH)], idxs[b], sem_i[b]).wait()

    def issue_gather(b):
      pltpu.async_copy(feats_hbm.at[idxs[b].at[0]], rows[b], sem_g[b])

    def wait_gather(b):
      pltpu.make_async_copy(
          feats_hbm.at[idxs[b].at[0]], rows[b], sem_g[b]).wait()

    def issue_scatter(b):
      pltpu.async_copy(rows[b], acc_sh.at[idxs[b].at[1]], sem_s[b], add=True)

    def wait_scatter(b):
      pltpu.make_async_copy(
          rows[b], acc_sh.at[idxs[b].at[1]], sem_s[b]).wait()

    issue_idx(0, chunk_off(0))
    issue_idx(1, chunk_off(1))

    z16 = jnp.zeros((16,), jnp.float32)
    for r in range(_ZR):
      for c in range(_D // 16):
        zero_v[r, pl.ds(c * 16, 16)] = z16
    for i in range(rows_per_tile // _ZR):
      pltpu.async_copy(
          zero_v, acc_sh.at[pl.ds(sid * rows_per_tile + i * _ZR, _ZR)], sz)
    for i in range(rows_per_tile // _ZR):
      pltpu.make_async_copy(
          zero_v, acc_sh.at[pl.ds(sid * rows_per_tile, _ZR)], sz).wait()
    plsc.subcore_barrier()

    def half(s, b):
      b2 = 1 - b
      wait_idx(b)
      issue_gather(b)
      wait_gather(b)
      wait_scatter(b2)
      issue_idx(b2, chunk_off(s + 1))
      issue_scatter(b)

    wait_idx(0)
    issue_gather(0)
    wait_gather(0)
    issue_scatter(0)

    def pair(k, carry):
      s = 2 * k + 1
      half(s, 1)
      half(s + 1, 0)
      return carry
    lax.fori_loop(0, (_CHW - 2) // 2, pair, 0)

    wait_idx(1)
    issue_gather(1)
    wait_gather(1)
    wait_scatter(0)
    issue_scatter(1)
    wait_scatter(1)

    plsc.subcore_barrier()

    pltpu.sync_copy(
        acc_sh.at[pl.ds(sid * rows_per_tile, rows_per_tile)],
        out_hbm.at[cid, pl.ds(sid * rows_per_tile, rows_per_tile)])

  return agg


_agg = _make_agg()

_R = 1000


def _init_transform(x, w_t):
  def body(x_ref, w_ref, o_ref):
    o_ref[...] = jnp.dot(x_ref[...], w_ref[...],
                         preferred_element_type=jnp.float32)
  return pl.pallas_call(
      body,
      grid=(_N // _R,),
      in_specs=[
          pl.BlockSpec((_R, _D), lambda i: (i, 0)),
          pl.BlockSpec((_D, _D), lambda i: (0, 0)),
      ],
      out_specs=pl.BlockSpec((_R, _D), lambda i: (i, 0)),
      out_shape=jax.ShapeDtypeStruct((_N, _D), jnp.float32),
  )(x, w_t)


def _gru(parts, feats, wl_t, bl, wi_t, wh_t, bi, bh):
  def body(p_ref, f_ref, wl_ref, bl_ref, wi_ref, wh_ref, bi_ref, bh_ref,
           o_ref):
    agg = p_ref[0] + p_ref[1]
    f = f_ref[...]
    h = jnp.dot(agg, wl_ref[...], preferred_element_type=jnp.float32) + bl_ref[...]
    gi = jnp.dot(h, wi_ref[...], preferred_element_type=jnp.float32) + bi_ref[...]
    gh = jnp.dot(f, wh_ref[...], preferred_element_type=jnp.float32) + bh_ref[...]
    r = jax.nn.sigmoid(gi[:, :_D] + gh[:, :_D])
    z = jax.nn.sigmoid(gi[:, _D:2 * _D] + gh[:, _D:2 * _D])
    n = jnp.tanh(gi[:, 2 * _D:] + r * gh[:, 2 * _D:])
    o_ref[...] = (1.0 - z) * n + z * f

  return pl.pallas_call(
      body,
      grid=(_N // _R,),
      in_specs=[
          pl.BlockSpec((_NC, _R, _D), lambda i: (0, i, 0)),
          pl.BlockSpec((_R, _D), lambda i: (i, 0)),
          pl.BlockSpec((_D, _D), lambda i: (0, 0)),
          pl.BlockSpec((1, _D), lambda i: (0, 0)),
          pl.BlockSpec((_D, 3 * _D), lambda i: (0, 0)),
          pl.BlockSpec((_D, 3 * _D), lambda i: (0, 0)),
          pl.BlockSpec((1, 3 * _D), lambda i: (0, 0)),
          pl.BlockSpec((1, 3 * _D), lambda i: (0, 0)),
      ],
      out_specs=pl.BlockSpec((_R, _D), lambda i: (i, 0)),
      out_shape=jax.ShapeDtypeStruct((_N, _D), jnp.float32),
  )(parts, feats, wl_t, bl, wi_t, wh_t, bi, bh)


def kernel(x, edge_index, batch_size, W_init,
           W_lin0, b_lin0, W_ih0, W_hh0, b_ih0, b_hh0,
           W_lin1, b_lin1, W_ih1, W_hh1, b_ih1, b_hh1):
  src = edge_index[0].astype(jnp.int32)
  dst = edge_index[1].astype(jnp.int32)

  npadgap = _NPAD - _N
  pad = _EPAD - _E
  pad_i = jnp.arange(pad, dtype=jnp.int32)
  ed = jnp.concatenate(
      [jnp.stack([src, dst]),
       jnp.stack([pad_i % _N, _N + pad_i % npadgap])], axis=1)

  feats = _init_transform(x, W_init.T)

  parts = _agg(feats, ed)
  feats = _gru(parts, feats, W_lin0.T, b_lin0.reshape(1, -1),
               W_ih0.T, W_hh0.T, b_ih0.reshape(1, -1), b_hh0.reshape(1, -1))

  parts = _agg(feats, ed)
  feats = _gru(parts, feats, W_lin1.T, b_lin1.reshape(1, -1),
               W_ih1.T, W_hh1.T, b_ih1.reshape(1, -1), b_hh1.reshape(1, -1))

  return feats.reshape(_BATCH, -1, _D)


# speedup vs baseline: 8.2283x; 1.0165x over previous
import functools

import jax
import jax.numpy as jnp
from jax import lax
from jax.experimental import pallas as pl
from jax.experimental.pallas import tpu as pltpu
from jax.experimental.pallas import tpu_sc as plsc

_N = 10000
_BATCH = 100
_E = 320000
_D = 128
_NPAD = 10240

_NC = 2
_NS = 16
_NW = _NC * _NS
_CH = 80
_CHW = 128
_EPAD = _CHW * _CH * _NW


def _make_agg():
  mesh = plsc.VectorSubcoreMesh(core_axis_name="c", subcore_axis_name="s")
  rows_per_tile = _NPAD // _NS
  _ZR = 32

  @functools.partial(
      pl.kernel,
      mesh=mesh,
      out_type=jax.ShapeDtypeStruct((_NC, _NPAD, _D), jnp.float32),
      scratch_types=(
          [pltpu.VMEM((_CH,), jnp.int32) for _ in range(8)]
          + [pltpu.VMEM((_CH, _D), jnp.float32) for _ in range(4)]
          + [pltpu.VMEM((_ZR, _D), jnp.float32)]
          + [pltpu.VMEM_SHARED((_NPAD, _D), jnp.float32)]
          + [pltpu.SemaphoreType.DMA for _ in range(13)]
      ),
  )
  def agg(feats_hbm, src_hbm, dst_hbm, out_hbm,
          sv0, sv1, sv2, sv3, dv0, dv1, dv2, dv3,
          rows0, rows1, rows2, rows3, zero_v, acc_sh,
          si0, si1, si2, si3, sg0, sg1, sg2, sg3, ss0, ss1, ss2, ss3, sz):
    cid = lax.axis_index("c")
    sid = lax.axis_index("s")
    wid = sid * _NC + cid

    srcs = (sv0, sv1, sv2, sv3)
    dsts = (dv0, dv1, dv2, dv3)
    rows = (rows0, rows1, rows2, rows3)
    sem_i = (si0, si1, si2, si3)
    sem_g = (sg0, sg1, sg2, sg3)
    sem_s = (ss0, ss1, ss2, ss3)

    def chunk_off(s):
      return (wid + s * _NW) * _CH

    def issue_idx(b, off):
      pltpu.async_copy(src_hbm.at[pl.ds(off, _CH)], srcs[b], sem_i[b])
      pltpu.async_copy(dst_hbm.at[pl.ds(off, _CH)], dsts[b], sem_i[b])

    def wait_idx(b):
      pltpu.make_async_copy(
          src_hbm.at[pl.ds(0, _CH)], srcs[b], sem_i[b]).wait()
      pltpu.make_async_copy(
          dst_hbm.at[pl.ds(0, _CH)], dsts[b], sem_i[b]).wait()

    def issue_gather(b):
      pltpu.async_copy(feats_hbm.at[srcs[b]], rows[b], sem_g[b])

    def wait_gather(b):
      pltpu.make_async_copy(
          feats_hbm.at[srcs[b]], rows[b], sem_g[b]).wait()

    def issue_scatter(b):
      pltpu.async_copy(rows[b], acc_sh.at[dsts[b]], sem_s[b], add=True)

    def wait_scatter(b):
      pltpu.make_async_copy(
          rows[b], acc_sh.at[dsts[b]], sem_s[b]).wait()

    issue_idx(0, chunk_off(0))
    issue_idx(1, chunk_off(1))
    issue_idx(2, chunk_off(2))

    z16 = jnp.zeros((16,), jnp.float32)
    for r in range(_ZR):
      for c in range(_D // 16):
        zero_v[r, pl.ds(c * 16, 16)] = z16
    for i in range(rows_per_tile // _ZR):
      pltpu.async_copy(
          zero_v, acc_sh.at[pl.ds(sid * rows_per_tile + i * _ZR, _ZR)], sz)
    for i in range(rows_per_tile // _ZR):
      pltpu.make_async_copy(
          zero_v, acc_sh.at[pl.ds(sid * rows_per_tile, _ZR)], sz).wait()
    plsc.subcore_barrier()

    def gather_scatter(b):
      wait_idx(b)
      issue_gather(b)
      wait_gather(b)
      issue_scatter(b)

    gather_scatter(0)
    issue_idx(3, chunk_off(3))
    gather_scatter(1)
    gather_scatter(2)

    def half(s, b):
      bf = (b + 1) % 4
      wait_scatter(bf)
      issue_idx(bf, chunk_off(s + 1))
      gather_scatter(b)

    def quad(k, carry):
      s = 4 * k + 3
      half(s, 3)
      half(s + 1, 0)
      half(s + 2, 1)
      half(s + 3, 2)
      return carry
    lax.fori_loop(0, (_CHW - 4) // 4, quad, 0)

    wait_scatter(0)
    gather_scatter(3)
    wait_scatter(1)
    wait_scatter(2)
    wait_scatter(3)

    plsc.subcore_barrier()

    pltpu.sync_copy(
        acc_sh.at[pl.ds(sid * rows_per_tile, rows_per_tile)],
        out_hbm.at[cid, pl.ds(sid * rows_per_tile, rows_per_tile)])

  return agg


_agg = _make_agg()

_R = 1000


def _init_transform(x, w_t):
  def body(x_ref, w_ref, o_ref):
    o_ref[...] = jnp.dot(x_ref[...], w_ref[...],
                         preferred_element_type=jnp.float32)
  return pl.pallas_call(
      body,
      grid=(_N // _R,),
      in_specs=[
          pl.BlockSpec((_R, _D), lambda i: (i, 0)),
          pl.BlockSpec((_D, _D), lambda i: (0, 0)),
      ],
      out_specs=pl.BlockSpec((_R, _D), lambda i: (i, 0)),
      out_shape=jax.ShapeDtypeStruct((_N, _D), jnp.float32),
  )(x, w_t)


def _gru(parts, feats, wl_t, bl, wi_t, wh_t, bi, bh):
  def body(p_ref, f_ref, wl_ref, bl_ref, wi_ref, wh_ref, bi_ref, bh_ref,
           o_ref):
    agg = p_ref[0] + p_ref[1]
    f = f_ref[...]
    h = jnp.dot(agg, wl_ref[...], preferred_element_type=jnp.float32) + bl_ref[...]
    gi = jnp.dot(h, wi_ref[...], preferred_element_type=jnp.float32) + bi_ref[...]
    gh = jnp.dot(f, wh_ref[...], preferred_element_type=jnp.float32) + bh_ref[...]
    r = jax.nn.sigmoid(gi[:, :_D] + gh[:, :_D])
    z = jax.nn.sigmoid(gi[:, _D:2 * _D] + gh[:, _D:2 * _D])
    n = jnp.tanh(gi[:, 2 * _D:] + r * gh[:, 2 * _D:])
    o_ref[...] = (1.0 - z) * n + z * f

  return pl.pallas_call(
      body,
      grid=(_N // _R,),
      in_specs=[
          pl.BlockSpec((_NC, _R, _D), lambda i: (0, i, 0)),
          pl.BlockSpec((_R, _D), lambda i: (i, 0)),
          pl.BlockSpec((_D, _D), lambda i: (0, 0)),
          pl.BlockSpec((1, _D), lambda i: (0, 0)),
          pl.BlockSpec((_D, 3 * _D), lambda i: (0, 0)),
          pl.BlockSpec((_D, 3 * _D), lambda i: (0, 0)),
          pl.BlockSpec((1, 3 * _D), lambda i: (0, 0)),
          pl.BlockSpec((1, 3 * _D), lambda i: (0, 0)),
      ],
      out_specs=pl.BlockSpec((_R, _D), lambda i: (i, 0)),
      out_shape=jax.ShapeDtypeStruct((_N, _D), jnp.float32),
  )(parts, feats, wl_t, bl, wi_t, wh_t, bi, bh)


def kernel(x, edge_index, batch_size, W_init,
           W_lin0, b_lin0, W_ih0, W_hh0, b_ih0, b_hh0,
           W_lin1, b_lin1, W_ih1, W_hh1, b_ih1, b_hh1):
  src = edge_index[0].astype(jnp.int32)
  dst = edge_index[1].astype(jnp.int32)

  npadgap = _NPAD - _N
  pad = _EPAD - _E
  pad_i = jnp.arange(pad, dtype=jnp.int32)
  src_p = jnp.concatenate([src, pad_i % _N])
  dst_p = jnp.concatenate([dst, _N + pad_i % npadgap])

  feats = _init_transform(x, W_init.T)

  parts = _agg(feats, src_p, dst_p)
  feats = _gru(parts, feats, W_lin0.T, b_lin0.reshape(1, -1),
               W_ih0.T, W_hh0.T, b_ih0.reshape(1, -1), b_hh0.reshape(1, -1))

  parts = _agg(feats, src_p, dst_p)
  feats = _gru(parts, feats, W_lin1.T, b_lin1.reshape(1, -1),
               W_ih1.T, W_hh1.T, b_ih1.reshape(1, -1), b_hh1.reshape(1, -1))

  return feats.reshape(_BATCH, -1, _D)
